# baseline (speedup 1.0000x reference)
import functools

import jax
import jax.numpy as jnp
from jax import lax
from jax.experimental import pallas as pl
from jax.experimental.pallas import tpu as pltpu
from jax.experimental.pallas import tpu_sc as plsc

_B = 16384
_D = 64
_NEG = 8
_NC = 2
_NS = 16
_NW = _NC * _NS
_CH = _B // _NW
_NCHUNK = 2 + _NEG
_NBUF = 3


_NH = 2
_DH = _D // _NH


def _gather_triplets(idx_i, idx_k, negs, table_halves):
    mesh = plsc.VectorSubcoreMesh(core_axis_name="c", subcore_axis_name="s")

    @functools.partial(
        pl.kernel,
        mesh=mesh,
        out_type=(
            jax.ShapeDtypeStruct((_B, _D), jnp.float32),
            jax.ShapeDtypeStruct((_B, _D), jnp.float32),
            jax.ShapeDtypeStruct((_B, _NEG, _D), jnp.float32),
        ),
        scratch_types=(
            [
                pltpu.VMEM((_CH,), jnp.int32),
                pltpu.VMEM((_CH,), jnp.int32),
                pltpu.VMEM((_NEG, _CH), jnp.int32),
            ]
            + [pltpu.VMEM((_CH, _DH), jnp.float32) for _ in range(_NBUF)]
            + [pltpu.SemaphoreType.DMA for _ in range(2 * _NBUF + 1)]
        ),
        compiler_params=pltpu.CompilerParams(
            use_tc_tiling_on_sc=False, needs_layout_passes=False),
    )
    def body(idx_i_hbm, idx_k_hbm, *rest):
        negs_hbm = rest[:_NEG]
        tables_hbm = rest[_NEG:_NEG + _NH]
        vi_hbm, vk_hbm, vj_hbm = rest[_NEG + _NH:_NEG + _NH + 3]
        sc = rest[_NEG + _NH + 3:]
        idxi_v, idxk_v, coln_v = sc[:3]
        bufs = sc[3:3 + _NBUF]
        gsem = sc[3 + _NBUF:3 + 2 * _NBUF]
        wsem = sc[3 + 2 * _NBUF:3 + 3 * _NBUF]
        isem = sc[3 + 3 * _NBUF]

        wid = lax.axis_index("s") * _NC + lax.axis_index("c")
        base = wid * _CH

        ih = [
            pltpu.async_copy(idx_i_hbm.at[pl.ds(base, _CH)], idxi_v, isem),
            pltpu.async_copy(idx_k_hbm.at[pl.ds(base, _CH)], idxk_v, isem),
        ] + [
            pltpu.async_copy(negs_hbm[c].at[pl.ds(base, _CH)],
                             coln_v.at[c], isem)
            for c in range(_NEG)
        ]
        for h in ih:
            h.wait()

        def out2(dst, h):
            return lambda buf, sem: pltpu.async_copy(
                buf, dst.at[pl.ds(base, _CH), pl.ds(h * _DH, _DH)], sem)

        def out3(c, h):
            return lambda buf, sem: pltpu.async_copy(
                buf, vj_hbm.at[pl.ds(base, _CH), c, pl.ds(h * _DH, _DH)],
                sem)

        chunks = []
        for h in range(_NH):
            chunks += [
                (h, idxi_v, out2(vi_hbm, h)),
                (h, idxk_v, out2(vk_hbm, h)),
            ] + [
                (h, coln_v.at[c], out3(c, h)) for c in range(_NEG)
            ]

        nch = len(chunks)
        gh, wh = {}, {}
        for t in range(nch + 1):
            if t < nch:
                b = t % _NBUF
                if t >= _NBUF:
                    wh[t - _NBUF].wait()
                h, idx_ref, _ = chunks[t]
                gh[t] = pltpu.async_copy(
                    tables_hbm[h].at[idx_ref], bufs[b], gsem[b])
            u = t - 1
            if 0 <= u < nch:
                b = u % _NBUF
                gh[u].wait()
                wh[u] = chunks[u][2](bufs[b], wsem[b])
        for u in range(nch - _NBUF, nch):
            wh[u].wait()

    return body(idx_i, idx_k, *negs, *table_halves)


def kernel(items, table):
    items = items.astype(jnp.int32)
    idx_i = items[:, 0]
    idx_k = items[:, 1]
    negs = [items[:, 2 + c] for c in range(_NEG)]
    halves = [table[:, h * _DH:(h + 1) * _DH] for h in range(_NH)]
    return _gather_triplets(idx_i, idx_k, negs, halves)


# speedup vs baseline: 2.0747x; 2.0747x over previous
import functools

import jax
import jax.numpy as jnp
from jax import lax
from jax.experimental import pallas as pl
from jax.experimental.pallas import tpu as pltpu
from jax.experimental.pallas import tpu_sc as plsc

_B = 16384
_D = 64
_NEG = 8
_NC = 2
_NS = 16
_NW = _NC * _NS
_CH = _B // _NW
_NCHUNK = 2 + _NEG
_NBUF = 3


def _gather_triplets(idx_i, idx_k, negs, table):
    mesh = plsc.VectorSubcoreMesh(core_axis_name="c", subcore_axis_name="s")

    @functools.partial(
        pl.kernel,
        mesh=mesh,
        out_type=(
            jax.ShapeDtypeStruct((_B, _D), jnp.float32),
            jax.ShapeDtypeStruct((_B, _D), jnp.float32),
            jax.ShapeDtypeStruct((_B, _NEG, _D), jnp.float32),
        ),
        scratch_types=(
            [
                pltpu.VMEM((_CH,), jnp.int32),
                pltpu.VMEM((_CH,), jnp.int32),
                pltpu.VMEM((_NEG, _CH), jnp.int32),
            ]
            + [pltpu.VMEM((_CH, _D), jnp.float32) for _ in range(_NBUF)]
            + [pltpu.SemaphoreType.DMA for _ in range(2 * _NBUF + 1)]
        ),
        compiler_params=pltpu.CompilerParams(
            use_tc_tiling_on_sc=False, needs_layout_passes=False),
    )
    def body(idx_i_hbm, idx_k_hbm, *rest):
        negs_hbm = rest[:_NEG]
        table_hbm, vi_hbm, vk_hbm, vj_hbm = rest[_NEG:_NEG + 4]
        idxi_v, idxk_v, coln_v = rest[_NEG + 4:_NEG + 7]
        bufs = rest[_NEG + 7:_NEG + 7 + _NBUF]
        gsem = rest[_NEG + 7 + _NBUF:_NEG + 7 + 2 * _NBUF]
        wsem = rest[_NEG + 7 + 2 * _NBUF:_NEG + 7 + 3 * _NBUF]
        isem = rest[_NEG + 7 + 3 * _NBUF]

        wid = lax.axis_index("s") * _NC + lax.axis_index("c")
        base = wid * _CH

        ih = [
            pltpu.async_copy(idx_i_hbm.at[pl.ds(base, _CH)], idxi_v, isem),
            pltpu.async_copy(idx_k_hbm.at[pl.ds(base, _CH)], idxk_v, isem),
        ] + [
            pltpu.async_copy(negs_hbm[c].at[pl.ds(base, _CH)],
                             coln_v.at[c], isem)
            for c in range(_NEG)
        ]
        for h in ih:
            h.wait()

        def out2(dst):
            return lambda buf, sem: pltpu.async_copy(
                buf, dst.at[pl.ds(base, _CH)], sem)

        def out3(c):
            return lambda buf, sem: pltpu.async_copy(
                buf, vj_hbm.at[pl.ds(base, _CH), c], sem)

        chunks = [
            (idxi_v, out2(vi_hbm)),
            (idxk_v, out2(vk_hbm)),
        ] + [
            (coln_v.at[c], out3(c)) for c in range(_NEG)
        ]

        gh, wh = {}, {}
        for t in range(_NCHUNK + 1):
            if t < _NCHUNK:
                b = t % _NBUF
                if t >= _NBUF:
                    wh[t - _NBUF].wait()
                gh[t] = pltpu.async_copy(
                    table_hbm.at[chunks[t][0]], bufs[b], gsem[b])
            u = t - 1
            if 0 <= u < _NCHUNK:
                b = u % _NBUF
                gh[u].wait()
                wh[u] = chunks[u][1](bufs[b], wsem[b])
        for u in range(_NCHUNK - _NBUF, _NCHUNK):
            wh[u].wait()

    return body(idx_i, idx_k, *negs, table)


def kernel(items, table):
    items = items.astype(jnp.int32)
    idx_i = items[:, 0]
    idx_k = items[:, 1]
    negs = [items[:, 2 + c] for c in range(_NEG)]
    return _gather_triplets(idx_i, idx_k, negs, table)
